# baseline (speedup 1.0000x reference)
import functools

import jax
import jax.numpy as jnp
from jax import lax
from jax.experimental import pallas as pl
from jax.experimental.pallas import tpu as pltpu
from jax.experimental.pallas import tpu_sc as plsc

_NC = 2
_NS = 16
_K = 128
_NG = 64
_EPS = 1e-5


def _sc_aggregate(h, src_p, dst_p, zeros, n, d, n_acc, epw, nchunk):
    mesh = plsc.VectorSubcoreMesh(core_axis_name="c", subcore_axis_name="s")
    zs = n_acc // _NS

    nh = nchunk // 2

    def body(h_hbm, src_hbm, dst_hbm, z_hbm, out0, out1,
             srcs_v, dsts_v, rows_v, acc_sh, sem0, sem1):
        cid = lax.axis_index("c")
        sid = lax.axis_index("s")
        wid = cid * _NS + sid
        row0 = sid * zs

        pltpu.sync_copy(dst_hbm.at[wid], dsts_v)
        pltpu.sync_copy(z_hbm, acc_sh.at[pl.ds(row0, zs)])
        plsc.subcore_barrier()

        for hh in range(2):
            c0 = hh * nh
            pltpu.sync_copy(src_hbm.at[wid, pl.ds(c0, nh)], srcs_v)
            pltpu.async_copy(h_hbm.at[srcs_v.at[0]], rows_v.at[0], sem0)

            def pair(p, carry):
                j = 2 * p
                pltpu.async_copy(h_hbm.at[srcs_v.at[j + 1]], rows_v.at[1],
                                 sem1)
                pltpu.make_async_copy(h_hbm.at[srcs_v.at[j]], rows_v.at[0],
                                      sem0).wait()
                pltpu.sync_copy(rows_v.at[0], acc_sh.at[dsts_v.at[c0 + j]],
                                add=True)

                @pl.when(j + 2 < nh)
                def _():
                    pltpu.async_copy(h_hbm.at[srcs_v.at[j + 2]],
                                     rows_v.at[0], sem0)

                pltpu.make_async_copy(h_hbm.at[srcs_v.at[j + 1]],
                                      rows_v.at[1], sem1).wait()
                pltpu.sync_copy(rows_v.at[1],
                                acc_sh.at[dsts_v.at[c0 + j + 1]], add=True)
                return carry

            lax.fori_loop(0, nh // 2, pair, 0)

        plsc.subcore_barrier()

        @pl.when(cid == 0)
        def _():
            pltpu.sync_copy(acc_sh.at[pl.ds(row0, zs)], out0.at[pl.ds(row0, zs)])

        @pl.when(cid == 1)
        def _():
            pltpu.sync_copy(acc_sh.at[pl.ds(row0, zs)], out1.at[pl.ds(row0, zs)])

    f = pl.kernel(
        body,
        out_type=(jax.ShapeDtypeStruct((n_acc, d), jnp.float32),
                  jax.ShapeDtypeStruct((n_acc, d), jnp.float32)),
        mesh=mesh,
        scratch_types=[
            pltpu.VMEM((nchunk // 2, _K), jnp.int32),
            pltpu.VMEM((nchunk, _K), jnp.int32),
            pltpu.VMEM((2, _K, d), jnp.float32),
            pltpu.VMEM_SHARED((n_acc, d), jnp.float32),
            pltpu.SemaphoreType.DMA,
            pltpu.SemaphoreType.DMA,
        ],
    )
    nw = _NC * _NS
    return f(h, src_p.reshape(nw, nchunk, _K), dst_p.reshape(nw, nchunk, _K),
             zeros)


def _mlp_bn(h, p0, p1, w1, b1, w2, b2, g, bt, relu_out, n, d, dh, bs):
    nblk = n // bs

    def body(h_r, p0_r, p1_r, w1_r, b1_r, w2_r, b2_r, g_r, bt_r, out_r,
             zpre, stats):
        i0 = pl.program_id(0)
        b = pl.program_id(1)

        @pl.when(i0 == 0)
        def _():
            @pl.when(b == 0)
            def _():
                stats[...] = jnp.zeros_like(stats)

            z = h_r[...] + p0_r[...] + p1_r[...]
            y = jnp.maximum(
                jnp.dot(z, w1_r[...], preferred_element_type=jnp.float32)
                + b1_r[...], 0.0)
            zo = (jnp.dot(y, w2_r[...], preferred_element_type=jnp.float32)
                  + b2_r[...])
            zpre[pl.ds(b * bs, bs), :] = zo
            stats[0:1, :] += jnp.sum(zo, axis=0, keepdims=True)
            stats[1:2, :] += jnp.sum(zo * zo, axis=0, keepdims=True)

        @pl.when(i0 == 1)
        def _():
            mu = stats[0:1, :] * (1.0 / n)
            var = stats[1:2, :] * (1.0 / n) - mu * mu
            sc = g_r[...] * lax.rsqrt(var + _EPS)
            o = (zpre[pl.ds(b * bs, bs), :] - mu) * sc + bt_r[...]
            if relu_out:
                o = jnp.maximum(o, 0.0)
            out_r[...] = o

    blk = lambda i0, b: (b * (1 - i0), 0)
    fixed = lambda i0, b: (0, 0)
    return pl.pallas_call(
        body,
        grid=(2, nblk),
        in_specs=[
            pl.BlockSpec((bs, d), blk),
            pl.BlockSpec((bs, d), blk),
            pl.BlockSpec((bs, d), blk),
            pl.BlockSpec((d, dh), fixed),
            pl.BlockSpec((1, dh), fixed),
            pl.BlockSpec((dh, d), fixed),
            pl.BlockSpec((1, d), fixed),
            pl.BlockSpec((1, d), fixed),
            pl.BlockSpec((1, d), fixed),
        ],
        out_specs=pl.BlockSpec((bs, d), lambda i0, b: (b, 0)),
        out_shape=jax.ShapeDtypeStruct((n, d), jnp.float32),
        scratch_shapes=[
            pltpu.VMEM((n, d), jnp.float32),
            pltpu.VMEM((8, d), jnp.float32),
        ],
    )(h, p0, p1, w1, b1.reshape(1, dh), w2, b2.reshape(1, d),
      g.reshape(1, d), bt.reshape(1, d))


def _pool(h, bf, p1w, p1b, p2w, p2b, g, bt, n, d, nhid, bs):
    nblk = n // bs

    def body(h_r, bf_r, p1w_r, p1b_r, p2w_r, p2b_r, g_r, bt_r, out_r,
             segs, cnts):
        b = pl.program_id(0)

        @pl.when(b == 0)
        def _():
            segs[...] = jnp.zeros_like(segs)
            cnts[...] = jnp.zeros_like(cnts)

        oh = (bf_r[...] == lax.broadcasted_iota(jnp.int32, (bs, _NG), 1)
              ).astype(jnp.float32)
        dn = (((0,), (0,)), ((), ()))
        segs[...] += lax.dot_general(oh, h_r[...], dn,
                                     preferred_element_type=jnp.float32)
        cnts[...] += lax.dot_general(oh, jnp.ones((bs, d), jnp.float32), dn,
                                     preferred_element_type=jnp.float32)

        @pl.when(b == nblk - 1)
        def _():
            hg = segs[...] / jnp.maximum(cnts[...], 1.0)
            y = jnp.maximum(
                jnp.dot(hg, p1w_r[...], preferred_element_type=jnp.float32)
                + p1b_r[...], 0.0)
            z = (jnp.dot(y, p2w_r[...], preferred_element_type=jnp.float32)
                 + p2b_r[...])
            mu = jnp.mean(z, axis=1, keepdims=True)
            zc = z - mu
            var = jnp.mean(zc * zc, axis=1, keepdims=True)
            out_r[...] = zc * lax.rsqrt(var + _EPS) * g_r[...] + bt_r[...]

    fixed = lambda b: (0, 0)
    return pl.pallas_call(
        body,
        grid=(nblk,),
        in_specs=[
            pl.BlockSpec((bs, d), lambda b: (b, 0)),
            pl.BlockSpec((bs, 1), lambda b: (b, 0)),
            pl.BlockSpec((d, nhid), fixed),
            pl.BlockSpec((1, nhid), fixed),
            pl.BlockSpec((nhid, d), fixed),
            pl.BlockSpec((1, d), fixed),
            pl.BlockSpec((1, d), fixed),
            pl.BlockSpec((1, d), fixed),
        ],
        out_specs=pl.BlockSpec((_NG, d), fixed),
        out_shape=jax.ShapeDtypeStruct((_NG, d), jnp.float32),
        scratch_shapes=[
            pltpu.VMEM((_NG, d), jnp.float32),
            pltpu.VMEM((_NG, d), jnp.float32),
        ],
    )(h, bf, p1w, p1b.reshape(1, nhid), p2w, p2b.reshape(1, d),
      g.reshape(1, d), bt.reshape(1, d))


def kernel(x, edge_index, batch, w1, b1, w2, b2, bng, bnb,
           p1w, p1b, p2w, p2b, ln2g, ln2b):
    n, d = x.shape
    e = edge_index.shape[1]
    nlayers = w1.shape[0]
    dh = w1.shape[2]
    nhid = p1w.shape[1]
    nw = _NC * _NS

    cpw = -(-e // (nw * _K))
    cpw = -(-cpw // 4) * 4
    e_pad = cpw * nw * _K
    epw = cpw * _K
    n_acc = -(-(n + 1) // (8 * _NS)) * (8 * _NS)

    src_p = jnp.concatenate(
        [edge_index[0], jnp.zeros((e_pad - e,), jnp.int32)])
    dst_p = jnp.concatenate(
        [edge_index[1], jnp.full((e_pad - e,), n, jnp.int32)])
    zeros = jnp.zeros((n_acc // _NS, d), jnp.float32)
    bf = batch.reshape(n, 1)
    bs = 1000 if n % 1000 == 0 else 8 * (n // 8)

    h = x
    for i in range(nlayers):
        p0, p1 = _sc_aggregate(h, src_p, dst_p, zeros, n, d, n_acc, epw, cpw)
        h = _mlp_bn(h, p0, p1, w1[i], b1[i], w2[i], b2[i], bng[i], bnb[i],
                    i < nlayers - 1, n, d, dh, bs)
    return _pool(h, bf, p1w, p1b, p2w, p2b, ln2g, ln2b, n, d, nhid, bs)


# speedup vs baseline: 1.7490x; 1.7490x over previous
import functools

import jax
import jax.numpy as jnp
from jax import lax
from jax.experimental import pallas as pl
from jax.experimental.pallas import tpu as pltpu
from jax.experimental.pallas import tpu_sc as plsc

_NC = 2
_NS = 16
_K = 128
_NG = 64
_EPS = 1e-5


def _sc_aggregate(h, src_sl, dst_sl, zeros, n, d, n_acc, cpw0, cpw1, cpw_max):
    mesh = plsc.VectorSubcoreMesh(core_axis_name="c", subcore_axis_name="s")
    zs = n_acc // _NS

    def body(h_hbm, src_hbm, dst_hbm, z_hbm, out0, out1,
             is0, is1, dsts_v, rows_v, acc_sh, sem0, sem1, semi0, semi1):
        cid = lax.axis_index("c")
        sid = lax.axis_index("s")
        wid = cid * _NS + sid
        row0 = sid * zs
        nch = jnp.where(cid == 0, cpw0, cpw1)

        pltpu.sync_copy(dst_hbm.at[wid], dsts_v)
        pltpu.sync_copy(z_hbm, acc_sh.at[pl.ds(row0, zs)])
        pltpu.async_copy(src_hbm.at[wid, 0], is0, semi0)
        pltpu.async_copy(src_hbm.at[wid, 1], is1, semi1)
        pltpu.make_async_copy(src_hbm.at[wid, 0], is0, semi0).wait()
        pltpu.async_copy(h_hbm.at[is0], rows_v.at[0], sem0)
        plsc.subcore_barrier()

        def pair(p, carry):
            j = 2 * p
            pltpu.make_async_copy(src_hbm.at[wid, 0], is1, semi1).wait()
            pltpu.async_copy(h_hbm.at[is1], rows_v.at[1], sem1)
            pltpu.make_async_copy(h_hbm.at[is0], rows_v.at[0], sem0).wait()

            @pl.when(j + 2 < nch)
            def _():
                pltpu.async_copy(src_hbm.at[wid, j + 2], is0, semi0)

            pltpu.sync_copy(rows_v.at[0], acc_sh.at[dsts_v.at[j]], add=True)

            @pl.when(j + 2 < nch)
            def _():
                pltpu.make_async_copy(src_hbm.at[wid, 0], is0, semi0).wait()
                pltpu.async_copy(h_hbm.at[is0], rows_v.at[0], sem0)

            pltpu.make_async_copy(h_hbm.at[is1], rows_v.at[1], sem1).wait()

            @pl.when(j + 3 < nch)
            def _():
                pltpu.async_copy(src_hbm.at[wid, j + 3], is1, semi1)

            pltpu.sync_copy(rows_v.at[1], acc_sh.at[dsts_v.at[j + 1]],
                            add=True)
            return carry

        lax.fori_loop(0, nch // 2, pair, 0)
        plsc.subcore_barrier()

        @pl.when(cid == 0)
        def _():
            pltpu.sync_copy(acc_sh.at[pl.ds(row0, zs)], out0.at[pl.ds(row0, zs)])

        @pl.when(cid == 1)
        def _():
            pltpu.sync_copy(acc_sh.at[pl.ds(row0, zs)], out1.at[pl.ds(row0, zs)])

    f = pl.kernel(
        body,
        out_type=(jax.ShapeDtypeStruct((n_acc, d), jnp.float32),
                  jax.ShapeDtypeStruct((n_acc, d), jnp.float32)),
        mesh=mesh,
        scratch_types=[
            pltpu.VMEM((_K,), jnp.int32),
            pltpu.VMEM((_K,), jnp.int32),
            pltpu.VMEM((cpw_max, _K), jnp.int32),
            pltpu.VMEM((2, _K, d), jnp.float32),
            pltpu.VMEM_SHARED((n_acc, d), jnp.float32),
            pltpu.SemaphoreType.DMA,
            pltpu.SemaphoreType.DMA,
            pltpu.SemaphoreType.DMA,
            pltpu.SemaphoreType.DMA,
        ],
    )
    return f(h, src_sl, dst_sl, zeros)


def _mlp_bn(h, p0, p1, w1, b1, w2, b2, g, bt, relu_out, n, d, dh, bs):
    nblk = n // bs

    def body(h_r, p0_r, p1_r, w1_r, b1_r, w2_r, b2_r, g_r, bt_r, out_r,
             zpre, stats):
        i0 = pl.program_id(0)
        b = pl.program_id(1)

        @pl.when(i0 == 0)
        def _():
            @pl.when(b == 0)
            def _():
                stats[...] = jnp.zeros_like(stats)

            z = h_r[...] + p0_r[...] + p1_r[...]
            y = jnp.maximum(
                jnp.dot(z, w1_r[...], preferred_element_type=jnp.float32)
                + b1_r[...], 0.0)
            zo = (jnp.dot(y, w2_r[...], preferred_element_type=jnp.float32)
                  + b2_r[...])
            zpre[pl.ds(b * bs, bs), :] = zo
            stats[0:1, :] += jnp.sum(zo, axis=0, keepdims=True)
            stats[1:2, :] += jnp.sum(zo * zo, axis=0, keepdims=True)

        @pl.when(i0 == 1)
        def _():
            mu = stats[0:1, :] * (1.0 / n)
            var = stats[1:2, :] * (1.0 / n) - mu * mu
            sc = g_r[...] * lax.rsqrt(var + _EPS)
            o = (zpre[pl.ds(b * bs, bs), :] - mu) * sc + bt_r[...]
            if relu_out:
                o = jnp.maximum(o, 0.0)
            out_r[...] = o

    blk = lambda i0, b: (b * (1 - i0), 0)
    fixed = lambda i0, b: (0, 0)
    return pl.pallas_call(
        body,
        grid=(2, nblk),
        in_specs=[
            pl.BlockSpec((bs, d), blk),
            pl.BlockSpec((bs, d), blk),
            pl.BlockSpec((bs, d), blk),
            pl.BlockSpec((d, dh), fixed),
            pl.BlockSpec((1, dh), fixed),
            pl.BlockSpec((dh, d), fixed),
            pl.BlockSpec((1, d), fixed),
            pl.BlockSpec((1, d), fixed),
            pl.BlockSpec((1, d), fixed),
        ],
        out_specs=pl.BlockSpec((bs, d), lambda i0, b: (b, 0)),
        out_shape=jax.ShapeDtypeStruct((n, d), jnp.float32),
        scratch_shapes=[
            pltpu.VMEM((n, d), jnp.float32),
            pltpu.VMEM((8, d), jnp.float32),
        ],
    )(h, p0, p1, w1, b1.reshape(1, dh), w2, b2.reshape(1, d),
      g.reshape(1, d), bt.reshape(1, d))


def _pool(h, bf, p1w, p1b, p2w, p2b, g, bt, n, d, nhid, bs):
    nblk = n // bs

    def body(h_r, bf_r, p1w_r, p1b_r, p2w_r, p2b_r, g_r, bt_r, out_r,
             segs, cnts):
        b = pl.program_id(0)

        @pl.when(b == 0)
        def _():
            segs[...] = jnp.zeros_like(segs)
            cnts[...] = jnp.zeros_like(cnts)

        oh = (bf_r[...] == lax.broadcasted_iota(jnp.int32, (bs, _NG), 1)
              ).astype(jnp.float32)
        dn = (((0,), (0,)), ((), ()))
        segs[...] += lax.dot_general(oh, h_r[...], dn,
                                     preferred_element_type=jnp.float32)
        cnts[...] += lax.dot_general(oh, jnp.ones((bs, d), jnp.float32), dn,
                                     preferred_element_type=jnp.float32)

        @pl.when(b == nblk - 1)
        def _():
            hg = segs[...] / jnp.maximum(cnts[...], 1.0)
            y = jnp.maximum(
                jnp.dot(hg, p1w_r[...], preferred_element_type=jnp.float32)
                + p1b_r[...], 0.0)
            z = (jnp.dot(y, p2w_r[...], preferred_element_type=jnp.float32)
                 + p2b_r[...])
            mu = jnp.mean(z, axis=1, keepdims=True)
            zc = z - mu
            var = jnp.mean(zc * zc, axis=1, keepdims=True)
            out_r[...] = zc * lax.rsqrt(var + _EPS) * g_r[...] + bt_r[...]

    fixed = lambda b: (0, 0)
    return pl.pallas_call(
        body,
        grid=(nblk,),
        in_specs=[
            pl.BlockSpec((bs, d), lambda b: (b, 0)),
            pl.BlockSpec((bs, 1), lambda b: (b, 0)),
            pl.BlockSpec((d, nhid), fixed),
            pl.BlockSpec((1, nhid), fixed),
            pl.BlockSpec((nhid, d), fixed),
            pl.BlockSpec((1, d), fixed),
            pl.BlockSpec((1, d), fixed),
            pl.BlockSpec((1, d), fixed),
        ],
        out_specs=pl.BlockSpec((_NG, d), fixed),
        out_shape=jax.ShapeDtypeStruct((_NG, d), jnp.float32),
        scratch_shapes=[
            pltpu.VMEM((_NG, d), jnp.float32),
            pltpu.VMEM((_NG, d), jnp.float32),
        ],
    )(h, bf, p1w, p1b.reshape(1, nhid), p2w, p2b.reshape(1, d),
      g.reshape(1, d), bt.reshape(1, d))


def kernel(x, edge_index, batch, w1, b1, w2, b2, bng, bnb,
           p1w, p1b, p2w, p2b, ln2g, ln2b):
    n, d = x.shape
    e = edge_index.shape[1]
    nlayers = w1.shape[0]
    dh = w1.shape[2]
    nhid = p1w.shape[1]
    nw = _NC * _NS

    f0 = 0.78
    cpw0 = max(2, int(round(e * f0 / (_NS * _K * 2))) * 2)
    e0 = _NS * cpw0 * _K
    cpw1 = max(2, -(-(e - e0) // (_NS * _K * 2)) * 2) if e > e0 else 2
    cpw_max = max(cpw0, cpw1)
    e_pad = e0 + _NS * cpw1 * _K
    n_acc = -(-(n + 1) // (8 * _NS)) * (8 * _NS)

    src_pad = jnp.concatenate(
        [edge_index[0], jnp.zeros((e_pad - e,), jnp.int32)])
    dst_pad = jnp.concatenate(
        [edge_index[1], jnp.full((e_pad - e,), n, jnp.int32)])
    src_sl = jnp.zeros((nw, cpw_max, _K), jnp.int32)
    dst_sl = jnp.full((nw, cpw_max, _K), n, jnp.int32)
    src_sl = src_sl.at[:_NS, :cpw0].set(src_pad[:e0].reshape(_NS, cpw0, _K))
    dst_sl = dst_sl.at[:_NS, :cpw0].set(dst_pad[:e0].reshape(_NS, cpw0, _K))
    src_sl = src_sl.at[_NS:, :cpw1].set(src_pad[e0:].reshape(_NS, cpw1, _K))
    dst_sl = dst_sl.at[_NS:, :cpw1].set(dst_pad[e0:].reshape(_NS, cpw1, _K))
    zeros = jnp.zeros((n_acc // _NS, d), jnp.float32)
    bf = batch.reshape(n, 1)
    bs = 1000 if n % 1000 == 0 else 8 * (n // 8)

    h = x
    for i in range(nlayers):
        p0, p1 = _sc_aggregate(h, src_sl, dst_sl, zeros, n, d, n_acc,
                               cpw0, cpw1, cpw_max)
        h = _mlp_bn(h, p0, p1, w1[i], b1[i], w2[i], b2[i], bng[i], bnb[i],
                    i < nlayers - 1, n, d, dh, bs)
    return _pool(h, bf, p1w, p1b, p2w, p2b, ln2g, ln2b, n, d, nhid, bs)


# speedup vs baseline: 1.7646x; 1.0090x over previous
import functools

import jax
import jax.numpy as jnp
from jax import lax
from jax.experimental import pallas as pl
from jax.experimental.pallas import tpu as pltpu
from jax.experimental.pallas import tpu_sc as plsc

_NC = 2
_NS = 16
_K = 128
_NG = 64
_EPS = 1e-5


def _sc_aggregate(h, src_sl, dst_sl, zeros, n, d, n_acc, cpw0, cpw1, cpw_max):
    mesh = plsc.VectorSubcoreMesh(core_axis_name="c", subcore_axis_name="s")
    zs = n_acc // _NS

    def body(h_hbm, src_hbm, dst_hbm, z_hbm, out0, out1,
             is0, is1, dsts_v, rows_v, acc_sh, sem0, sem1, semi0, semi1):
        cid = lax.axis_index("c")
        sid = lax.axis_index("s")
        wid = cid * _NS + sid
        row0 = sid * zs
        nch = jnp.where(cid == 0, cpw0, cpw1)

        pltpu.sync_copy(dst_hbm.at[wid], dsts_v)
        pltpu.sync_copy(z_hbm, acc_sh.at[pl.ds(row0, zs)])
        pltpu.async_copy(src_hbm.at[wid, 0], is0, semi0)
        pltpu.async_copy(src_hbm.at[wid, 1], is1, semi1)
        pltpu.make_async_copy(src_hbm.at[wid, 0], is0, semi0).wait()
        pltpu.async_copy(h_hbm.at[is0], rows_v.at[0], sem0)
        plsc.subcore_barrier()

        def pair(p, carry):
            j = 2 * p
            pltpu.make_async_copy(src_hbm.at[wid, 0], is1, semi1).wait()
            pltpu.async_copy(h_hbm.at[is1], rows_v.at[1], sem1)
            pltpu.make_async_copy(h_hbm.at[is0], rows_v.at[0], sem0).wait()

            @pl.when(j + 2 < nch)
            def _():
                pltpu.async_copy(src_hbm.at[wid, j + 2], is0, semi0)

            pltpu.sync_copy(rows_v.at[0], acc_sh.at[dsts_v.at[j]], add=True)

            @pl.when(j + 2 < nch)
            def _():
                pltpu.make_async_copy(src_hbm.at[wid, 0], is0, semi0).wait()
                pltpu.async_copy(h_hbm.at[is0], rows_v.at[0], sem0)

            pltpu.make_async_copy(h_hbm.at[is1], rows_v.at[1], sem1).wait()

            @pl.when(j + 3 < nch)
            def _():
                pltpu.async_copy(src_hbm.at[wid, j + 3], is1, semi1)

            pltpu.sync_copy(rows_v.at[1], acc_sh.at[dsts_v.at[j + 1]],
                            add=True)
            return carry

        lax.fori_loop(0, nch // 2, pair, 0)
        plsc.subcore_barrier()

        @pl.when(cid == 0)
        def _():
            pltpu.sync_copy(acc_sh.at[pl.ds(row0, zs)], out0.at[pl.ds(row0, zs)])

        @pl.when(cid == 1)
        def _():
            pltpu.sync_copy(acc_sh.at[pl.ds(row0, zs)], out1.at[pl.ds(row0, zs)])

    f = pl.kernel(
        body,
        out_type=(jax.ShapeDtypeStruct((n_acc, d), jnp.float32),
                  jax.ShapeDtypeStruct((n_acc, d), jnp.float32)),
        mesh=mesh,
        scratch_types=[
            pltpu.VMEM((_K,), jnp.int32),
            pltpu.VMEM((_K,), jnp.int32),
            pltpu.VMEM((cpw_max, _K), jnp.int32),
            pltpu.VMEM((2, _K, d), jnp.float32),
            pltpu.VMEM_SHARED((n_acc, d), jnp.float32),
            pltpu.SemaphoreType.DMA,
            pltpu.SemaphoreType.DMA,
            pltpu.SemaphoreType.DMA,
            pltpu.SemaphoreType.DMA,
        ],
    )
    return f(h, src_sl, dst_sl, zeros)


def _mlp_bn(h, p0, p1, w1, b1, w2, b2, g, bt, relu_out, n, d, dh, bs):
    nblk = n // bs

    def body(h_r, p0_r, p1_r, w1_r, b1_r, w2_r, b2_r, g_r, bt_r, out_r,
             zpre, stats):
        i0 = pl.program_id(0)
        b = pl.program_id(1)

        @pl.when(i0 == 0)
        def _():
            @pl.when(b == 0)
            def _():
                stats[...] = jnp.zeros_like(stats)

            z = h_r[...] + p0_r[...] + p1_r[...]
            y = jnp.maximum(
                jnp.dot(z, w1_r[...], preferred_element_type=jnp.float32)
                + b1_r[...], 0.0)
            zo = (jnp.dot(y, w2_r[...], preferred_element_type=jnp.float32)
                  + b2_r[...])
            zpre[pl.ds(b * bs, bs), :] = zo
            stats[0:1, :] += jnp.sum(zo, axis=0, keepdims=True)
            stats[1:2, :] += jnp.sum(zo * zo, axis=0, keepdims=True)

        @pl.when(i0 == 1)
        def _():
            mu = stats[0:1, :] * (1.0 / n)
            var = stats[1:2, :] * (1.0 / n) - mu * mu
            sc = g_r[...] * lax.rsqrt(var + _EPS)
            o = (zpre[pl.ds(b * bs, bs), :] - mu) * sc + bt_r[...]
            if relu_out:
                o = jnp.maximum(o, 0.0)
            out_r[...] = o

    blk = lambda i0, b: (b * (1 - i0), 0)
    fixed = lambda i0, b: (0, 0)
    return pl.pallas_call(
        body,
        grid=(2, nblk),
        in_specs=[
            pl.BlockSpec((bs, d), blk),
            pl.BlockSpec((bs, d), blk),
            pl.BlockSpec((bs, d), blk),
            pl.BlockSpec((d, dh), fixed),
            pl.BlockSpec((1, dh), fixed),
            pl.BlockSpec((dh, d), fixed),
            pl.BlockSpec((1, d), fixed),
            pl.BlockSpec((1, d), fixed),
            pl.BlockSpec((1, d), fixed),
        ],
        out_specs=pl.BlockSpec((bs, d), lambda i0, b: (b, 0)),
        out_shape=jax.ShapeDtypeStruct((n, d), jnp.float32),
        scratch_shapes=[
            pltpu.VMEM((n, d), jnp.float32),
            pltpu.VMEM((8, d), jnp.float32),
        ],
    )(h, p0, p1, w1, b1.reshape(1, dh), w2, b2.reshape(1, d),
      g.reshape(1, d), bt.reshape(1, d))


def _pool(h, bf, p1w, p1b, p2w, p2b, g, bt, n, d, nhid, bs):
    nblk = n // bs

    def body(h_r, bf_r, p1w_r, p1b_r, p2w_r, p2b_r, g_r, bt_r, out_r,
             segs, cnts):
        b = pl.program_id(0)

        @pl.when(b == 0)
        def _():
            segs[...] = jnp.zeros_like(segs)
            cnts[...] = jnp.zeros_like(cnts)

        oh = (bf_r[...] == lax.broadcasted_iota(jnp.int32, (bs, _NG), 1)
              ).astype(jnp.float32)
        dn = (((0,), (0,)), ((), ()))
        segs[...] += lax.dot_general(oh, h_r[...], dn,
                                     preferred_element_type=jnp.float32)
        cnts[...] += lax.dot_general(oh, jnp.ones((bs, d), jnp.float32), dn,
                                     preferred_element_type=jnp.float32)

        @pl.when(b == nblk - 1)
        def _():
            hg = segs[...] / jnp.maximum(cnts[...], 1.0)
            y = jnp.maximum(
                jnp.dot(hg, p1w_r[...], preferred_element_type=jnp.float32)
                + p1b_r[...], 0.0)
            z = (jnp.dot(y, p2w_r[...], preferred_element_type=jnp.float32)
                 + p2b_r[...])
            mu = jnp.mean(z, axis=1, keepdims=True)
            zc = z - mu
            var = jnp.mean(zc * zc, axis=1, keepdims=True)
            out_r[...] = zc * lax.rsqrt(var + _EPS) * g_r[...] + bt_r[...]

    fixed = lambda b: (0, 0)
    return pl.pallas_call(
        body,
        grid=(nblk,),
        in_specs=[
            pl.BlockSpec((bs, d), lambda b: (b, 0)),
            pl.BlockSpec((bs, 1), lambda b: (b, 0)),
            pl.BlockSpec((d, nhid), fixed),
            pl.BlockSpec((1, nhid), fixed),
            pl.BlockSpec((nhid, d), fixed),
            pl.BlockSpec((1, d), fixed),
            pl.BlockSpec((1, d), fixed),
            pl.BlockSpec((1, d), fixed),
        ],
        out_specs=pl.BlockSpec((_NG, d), fixed),
        out_shape=jax.ShapeDtypeStruct((_NG, d), jnp.float32),
        scratch_shapes=[
            pltpu.VMEM((_NG, d), jnp.float32),
            pltpu.VMEM((_NG, d), jnp.float32),
        ],
    )(h, bf, p1w, p1b.reshape(1, nhid), p2w, p2b.reshape(1, d),
      g.reshape(1, d), bt.reshape(1, d))


def kernel(x, edge_index, batch, w1, b1, w2, b2, bng, bnb,
           p1w, p1b, p2w, p2b, ln2g, ln2b):
    n, d = x.shape
    e = edge_index.shape[1]
    nlayers = w1.shape[0]
    dh = w1.shape[2]
    nhid = p1w.shape[1]
    nw = _NC * _NS

    f0 = 0.84
    cpw0 = max(2, min(124, int(round(e * f0 / (_NS * _K * 2))) * 2))
    e0 = _NS * cpw0 * _K
    cpw1 = max(2, -(-(e - e0) // (_NS * _K * 2)) * 2) if e > e0 else 2
    cpw_max = max(cpw0, cpw1)
    e_pad = e0 + _NS * cpw1 * _K
    n_acc = -(-(n + 1) // (8 * _NS)) * (8 * _NS)

    src_pad = jnp.concatenate(
        [edge_index[0], jnp.zeros((e_pad - e,), jnp.int32)])
    dst_pad = jnp.concatenate(
        [edge_index[1], jnp.full((e_pad - e,), n, jnp.int32)])
    src_sl = jnp.zeros((nw, cpw_max, _K), jnp.int32)
    dst_sl = jnp.full((nw, cpw_max, _K), n, jnp.int32)
    src_sl = src_sl.at[:_NS, :cpw0].set(src_pad[:e0].reshape(_NS, cpw0, _K))
    dst_sl = dst_sl.at[:_NS, :cpw0].set(dst_pad[:e0].reshape(_NS, cpw0, _K))
    src_sl = src_sl.at[_NS:, :cpw1].set(src_pad[e0:].reshape(_NS, cpw1, _K))
    dst_sl = dst_sl.at[_NS:, :cpw1].set(dst_pad[e0:].reshape(_NS, cpw1, _K))
    zeros = jnp.zeros((n_acc // _NS, d), jnp.float32)
    bf = batch.reshape(n, 1)
    bs = 1000 if n % 1000 == 0 else 8 * (n // 8)

    h = x
    for i in range(nlayers):
        p0, p1 = _sc_aggregate(h, src_sl, dst_sl, zeros, n, d, n_acc,
                               cpw0, cpw1, cpw_max)
        h = _mlp_bn(h, p0, p1, w1[i], b1[i], w2[i], b2[i], bng[i], bnb[i],
                    i < nlayers - 1, n, d, dh, bs)
    return _pool(h, bf, p1w, p1b, p2w, p2b, ln2g, ln2b, n, d, nhid, bs)
